# baseline (speedup 1.0000x reference)
import functools

import jax
import jax.numpy as jnp
from jax import lax
from jax.experimental import pallas as pl
from jax.experimental.pallas import tpu as pltpu
from jax.experimental.pallas import tpu_sc as plsc

N = 10000
NP = 10240
D = 128
NC = 2
NS = 16
NW = NC * NS
C = 128
ROWS_PER_TILE = NP // NS

_mesh = plsc.VectorSubcoreMesh(core_axis_name="c", subcore_axis_name="s")


def _make_spmm(nch):

    hc = nch // 2

    @functools.partial(
        pl.kernel,
        out_type=jax.ShapeDtypeStruct((NC, NP, D), jnp.float32),
        mesh=_mesh,
        scratch_types=[
            pltpu.VMEM((hc, C), jnp.int32),
            pltpu.VMEM((nch, C), jnp.int32),
            pltpu.VMEM((C, D), jnp.float32),
            pltpu.VMEM((C, D), jnp.float32),
            pltpu.VMEM_SHARED((NP, D), jnp.float32),
            pltpu.SemaphoreType.DMA,
            pltpu.SemaphoreType.DMA,
            pltpu.SemaphoreType.DMA,
            pltpu.SemaphoreType.DMA,
            pltpu.SemaphoreType.DMA,
            pltpu.SemaphoreType.DMA,
        ],
    )
    def spmm(hp_hbm, src_hbm, dst_hbm, zeros_hbm, out_hbm,
             srcs, dsts, b0, b1, acc, gs0a, gs0b, gs1a, gs1b, ss0, ss1):

        def gather_start(j, buf, sa, sb):
            da = pltpu.async_copy(hp_hbm.at[srcs.at[j, pl.ds(0, C // 2)]],
                                  buf.at[pl.ds(0, C // 2)], sa)
            db = pltpu.async_copy(hp_hbm.at[srcs.at[j, pl.ds(C // 2, C // 2)]],
                                  buf.at[pl.ds(C // 2, C // 2)], sb)
            return da, db

        def gather_wait(j, buf, sa, sb):
            pltpu.make_async_copy(hp_hbm.at[srcs.at[j, pl.ds(0, C // 2)]],
                                  buf.at[pl.ds(0, C // 2)], sa).wait()
            pltpu.make_async_copy(hp_hbm.at[srcs.at[j, pl.ds(C // 2, C // 2)]],
                                  buf.at[pl.ds(C // 2, C // 2)], sb).wait()
        c = lax.axis_index("c")
        s = lax.axis_index("s")
        wid = s * NC + c
        r0 = s * ROWS_PER_TILE
        pltpu.async_copy(dst_hbm.at[pl.ds(wid * nch, nch)], dsts, gs1a)
        pltpu.sync_copy(zeros_hbm.at[pl.ds(r0, ROWS_PER_TILE)],
                        acc.at[pl.ds(r0, ROWS_PER_TILE)])
        pltpu.make_async_copy(dst_hbm.at[pl.ds(wid * nch, nch)], dsts, gs1a).wait()
        plsc.subcore_barrier()

        for h in range(2):
            c0 = h * hc
            pltpu.sync_copy(src_hbm.at[pl.ds(wid * nch + c0, hc)], srcs)
            gather_start(0, b0, gs0a, gs0b)

            def pair(j, carry):
                gather_start(j + 1, b1, gs1a, gs1b)
                gather_wait(j, b0, gs0a, gs0b)
                d_s0 = pltpu.async_copy(b0, acc.at[dsts.at[c0 + j]], ss0,
                                        add=True)
                gather_wait(j + 1, b1, gs1a, gs1b)
                d_s0.wait()
                gather_start(j + 2, b0, gs0a, gs0b)
                d_s1 = pltpu.async_copy(b1, acc.at[dsts.at[c0 + j + 1]], ss1,
                                        add=True)
                d_s1.wait()
                return carry

            lax.fori_loop(0, (hc - 2) // 2, lambda i, cr: pair(2 * i, cr), 0,
                          unroll=False)

            j = hc - 2
            gather_start(j + 1, b1, gs1a, gs1b)
            gather_wait(j, b0, gs0a, gs0b)
            d_s0 = pltpu.async_copy(b0, acc.at[dsts.at[c0 + j]], ss0, add=True)
            gather_wait(j + 1, b1, gs1a, gs1b)
            d_s1 = pltpu.async_copy(b1, acc.at[dsts.at[c0 + j + 1]], ss1,
                                    add=True)
            d_s0.wait()
            d_s1.wait()

        plsc.subcore_barrier()
        pltpu.sync_copy(acc.at[pl.ds(r0, ROWS_PER_TILE)],
                        out_hbm.at[c, pl.ds(r0, ROWS_PER_TILE)])

    return spmm


DW = 16


def _make_deg(nch):
    @functools.partial(
        pl.kernel,
        out_type=jax.ShapeDtypeStruct((NC, NP, DW), jnp.float32),
        mesh=_mesh,
        scratch_types=[
            pltpu.VMEM((nch, C), jnp.int32),
            pltpu.VMEM((C, DW), jnp.float32),
            pltpu.VMEM_SHARED((NP, DW), jnp.float32),
            pltpu.SemaphoreType.DMA,
            pltpu.SemaphoreType.DMA,
        ],
    )
    def deg(dst_hbm, ones_hbm, zeros_hbm, out_hbm, dsts, ones_b, acc, s0, s1):
        c = lax.axis_index("c")
        s = lax.axis_index("s")
        wid = s * NC + c
        r0 = s * ROWS_PER_TILE
        pltpu.async_copy(dst_hbm.at[pl.ds(wid * nch, nch)], dsts, s0)
        pltpu.sync_copy(zeros_hbm.at[pl.ds(r0, ROWS_PER_TILE)],
                        acc.at[pl.ds(r0, ROWS_PER_TILE)])
        pltpu.sync_copy(ones_hbm, ones_b)
        pltpu.make_async_copy(dst_hbm.at[pl.ds(wid * nch, nch)], dsts, s0).wait()
        plsc.subcore_barrier()

        pltpu.async_copy(ones_b, acc.at[dsts.at[0]], s1, add=True)

        def body(j, cr):
            pltpu.async_copy(ones_b, acc.at[dsts.at[j]], s1, add=True)
            pltpu.make_async_copy(ones_b, acc.at[dsts.at[0]], s1).wait()
            return cr

        lax.fori_loop(1, nch, body, 0, unroll=False)
        pltpu.make_async_copy(ones_b, acc.at[dsts.at[0]], s1).wait()

        plsc.subcore_barrier()
        pltpu.sync_copy(acc.at[pl.ds(r0, ROWS_PER_TILE)],
                        out_hbm.at[c, pl.ds(r0, ROWS_PER_TILE)])

    return deg


BR = 2000
_GRID = N // BR


def _l2n(h):
    return h / (jnp.sqrt(jnp.sum(h * h, axis=1, keepdims=True)) + 1e-8)


def _enc_body(x_ref, w1_ref, b1_ref, degp_ref, h_ref, hp_ref, a_ref):
    h = jnp.maximum(
        jnp.dot(x_ref[...], w1_ref[...], preferred_element_type=jnp.float32)
        + b1_ref[...], 0.0)
    h = _l2n(h)
    a = lax.rsqrt(degp_ref[0, :, 0:1] + degp_ref[1, :, 0:1] + 1.0)
    h_ref[...] = h
    a_ref[...] = jnp.broadcast_to(a, h.shape)
    hp_ref[...] = h * a


def _gru_body(p_ref, h_ref, a_ref, wih_ref, whh_ref, bih_ref, bhh_ref, z_ref,
              hn_ref, hpn_ref, zn_ref):
    a = a_ref[...]
    h = h_ref[...]
    agg = (p_ref[0] + p_ref[1]) * a
    gi = jnp.dot(agg, wih_ref[...], preferred_element_type=jnp.float32) + bih_ref[...]
    gh = jnp.dot(h, whh_ref[...], preferred_element_type=jnp.float32) + bhh_ref[...]
    r = jax.nn.sigmoid(gi[:, :D] + gh[:, :D])
    zg = jax.nn.sigmoid(gi[:, D:2 * D] + gh[:, D:2 * D])
    n = jnp.tanh(gi[:, 2 * D:] + r * gh[:, 2 * D:])
    hn = _l2n((1.0 - zg) * n + zg * h)
    hn_ref[...] = hn
    hpn_ref[...] = hn * a
    zn_ref[...] = jnp.maximum(z_ref[...], hn)


def _head_body(z_ref, w2_ref, b2_ref, out_ref):
    out_ref[...] = (
        jnp.dot(z_ref[...], w2_ref[...], preferred_element_type=jnp.float32)
        + b2_ref[...])


def _row_spec(width):
    return pl.BlockSpec((BR, width), lambda i: (i, 0))


def _full_spec(shape):
    return pl.BlockSpec(shape, lambda i: tuple(0 for _ in shape))


_encoder = pl.pallas_call(
    _enc_body,
    grid=(_GRID,),
    in_specs=[
        _row_spec(D),
        _full_spec((D, D)),
        _full_spec((1, D)),
        pl.BlockSpec((NC, BR, DW), lambda i: (0, i, 0)),
    ],
    out_specs=[_row_spec(D), _row_spec(D), _row_spec(D)],
    out_shape=[jax.ShapeDtypeStruct((N, D), jnp.float32)] * 3,
)

_gru = pl.pallas_call(
    _gru_body,
    grid=(_GRID,),
    in_specs=[
        pl.BlockSpec((NC, BR, D), lambda i: (0, i, 0)),
        _row_spec(D),
        _row_spec(D),
        _full_spec((D, 3 * D)),
        _full_spec((D, 3 * D)),
        _full_spec((1, 3 * D)),
        _full_spec((1, 3 * D)),
        _row_spec(D),
    ],
    out_specs=[_row_spec(D), _row_spec(D), _row_spec(D)],
    out_shape=[jax.ShapeDtypeStruct((N, D), jnp.float32)] * 3,
)

_head = pl.pallas_call(
    _head_body,
    grid=(_GRID,),
    in_specs=[_row_spec(D), _full_spec((D, D)), _full_spec((1, D))],
    out_specs=_row_spec(D),
    out_shape=jax.ShapeDtypeStruct((N, D), jnp.float32),
)


def kernel(x, edge_index, W1, b1, Wih, Whh, bih, bhh, W2, b2):
    E = edge_index.shape[1]
    nch = -(-E // (C * NW))
    nch += -nch % 16
    e_pad = C * NW * nch
    spmm = _make_spmm(nch)
    degk = _make_deg(nch)

    src = edge_index[0].astype(jnp.int32)
    dst = edge_index[1].astype(jnp.int32)
    pad = e_pad - E
    src_p = jnp.concatenate([src, jnp.zeros((pad,), jnp.int32)]).reshape(-1, C)
    dst_p = jnp.concatenate([dst, jnp.full((pad,), N, jnp.int32)]).reshape(-1, C)

    zeros_nd = jnp.zeros((NP, D), jnp.float32)
    ones_cw = jnp.ones((C, DW), jnp.float32)
    zeros_nw = jnp.zeros((NP, DW), jnp.float32)

    degp = degk(dst_p, ones_cw, zeros_nw)
    h, hp, a = _encoder(x, W1, b1.reshape(1, D), degp)
    z = h
    bih2 = bih.reshape(1, 3 * D)
    bhh2 = bhh.reshape(1, 3 * D)
    for _ in range(4):
        p = spmm(hp, src_p, dst_p, zeros_nd)
        h, hp, z = _gru(p, h, a, Wih, Whh, bih2, bhh2, z)
    return _head(z, W2, b2.reshape(1, D))


# speedup vs baseline: 1.3003x; 1.3003x over previous
import functools

import jax
import jax.numpy as jnp
from jax import lax
from jax.experimental import pallas as pl
from jax.experimental.pallas import tpu as pltpu
from jax.experimental.pallas import tpu_sc as plsc

N = 10000
NP = 10240
D = 128
NC = 2
NS = 16
NW = NC * NS
C = 128
ROWS_PER_TILE = NP // NS

_mesh = plsc.VectorSubcoreMesh(core_axis_name="c", subcore_axis_name="s")


def _make_spmm(nch):

    hc = nch // 2

    @functools.partial(
        pl.kernel,
        out_type=jax.ShapeDtypeStruct((NC, NP, D), jnp.float32),
        mesh=_mesh,
        scratch_types=[
            pltpu.VMEM((hc, C), jnp.int32),
            pltpu.VMEM((nch, C), jnp.int32),
            pltpu.VMEM((C, D), jnp.float32),
            pltpu.VMEM((C, D), jnp.float32),
            pltpu.VMEM_SHARED((NP, D), jnp.float32),
            pltpu.SemaphoreType.DMA,
            pltpu.SemaphoreType.DMA,
            pltpu.SemaphoreType.DMA,
            pltpu.SemaphoreType.DMA,
            pltpu.SemaphoreType.DMA,
            pltpu.SemaphoreType.DMA,
        ],
    )
    def spmm(hp_hbm, src_hbm, dst_hbm, zeros_hbm, out_hbm,
             srcs, dsts, b0, b1, acc, gs0a, gs0b, gs1a, gs1b, ss0, ss1):

        def gather_start(j, buf, sa, sb):
            da = pltpu.async_copy(hp_hbm.at[srcs.at[j, pl.ds(0, C // 2)]],
                                  buf.at[pl.ds(0, C // 2)], sa)
            db = pltpu.async_copy(hp_hbm.at[srcs.at[j, pl.ds(C // 2, C // 2)]],
                                  buf.at[pl.ds(C // 2, C // 2)], sb)
            return da, db

        def gather_wait(j, buf, sa, sb):
            pltpu.make_async_copy(hp_hbm.at[srcs.at[j, pl.ds(0, C // 2)]],
                                  buf.at[pl.ds(0, C // 2)], sa).wait()
            pltpu.make_async_copy(hp_hbm.at[srcs.at[j, pl.ds(C // 2, C // 2)]],
                                  buf.at[pl.ds(C // 2, C // 2)], sb).wait()
        c = lax.axis_index("c")
        s = lax.axis_index("s")
        wid = s * NC + c
        r0 = s * ROWS_PER_TILE
        pltpu.async_copy(dst_hbm.at[pl.ds(wid * nch, nch)], dsts, gs1a)
        pltpu.sync_copy(zeros_hbm.at[pl.ds(r0, ROWS_PER_TILE)],
                        acc.at[pl.ds(r0, ROWS_PER_TILE)])
        pltpu.make_async_copy(dst_hbm.at[pl.ds(wid * nch, nch)], dsts, gs1a).wait()
        plsc.subcore_barrier()

        for h in range(2):
            c0 = h * hc
            pltpu.sync_copy(src_hbm.at[pl.ds(wid * nch + c0, hc)], srcs)
            gather_start(0, b0, gs0a, gs0b)

            def pair(j, carry):
                gather_start(j + 1, b1, gs1a, gs1b)
                gather_wait(j, b0, gs0a, gs0b)
                d_s0 = pltpu.async_copy(b0, acc.at[dsts.at[c0 + j]], ss0,
                                        add=True)
                gather_wait(j + 1, b1, gs1a, gs1b)
                d_s0.wait()
                gather_start(j + 2, b0, gs0a, gs0b)
                d_s1 = pltpu.async_copy(b1, acc.at[dsts.at[c0 + j + 1]], ss1,
                                        add=True)
                d_s1.wait()
                return carry

            lax.fori_loop(0, (hc - 2) // 2, lambda i, cr: pair(2 * i, cr), 0,
                          unroll=False)

            j = hc - 2
            gather_start(j + 1, b1, gs1a, gs1b)
            gather_wait(j, b0, gs0a, gs0b)
            d_s0 = pltpu.async_copy(b0, acc.at[dsts.at[c0 + j]], ss0, add=True)
            gather_wait(j + 1, b1, gs1a, gs1b)
            d_s1 = pltpu.async_copy(b1, acc.at[dsts.at[c0 + j + 1]], ss1,
                                    add=True)
            d_s0.wait()
            d_s1.wait()

        plsc.subcore_barrier()
        pltpu.sync_copy(acc.at[pl.ds(r0, ROWS_PER_TILE)],
                        out_hbm.at[c, pl.ds(r0, ROWS_PER_TILE)])

    return spmm


DW = 128


def _make_deg(nch):
    @functools.partial(
        pl.kernel,
        out_type=jax.ShapeDtypeStruct((NC, NP, DW), jnp.float32),
        mesh=_mesh,
        scratch_types=[
            pltpu.VMEM((nch, C), jnp.int32),
            pltpu.VMEM((C, DW), jnp.float32),
            pltpu.VMEM_SHARED((NP, DW), jnp.float32),
            pltpu.SemaphoreType.DMA,
            pltpu.SemaphoreType.DMA,
        ],
    )
    def deg(dst_hbm, ones_hbm, zeros_hbm, out_hbm, dsts, ones_b, acc, s0, s1):
        c = lax.axis_index("c")
        s = lax.axis_index("s")
        wid = s * NC + c
        r0 = s * ROWS_PER_TILE
        pltpu.async_copy(dst_hbm.at[pl.ds(wid * nch, nch)], dsts, s0)
        pltpu.sync_copy(zeros_hbm.at[pl.ds(r0, ROWS_PER_TILE)],
                        acc.at[pl.ds(r0, ROWS_PER_TILE)])
        pltpu.sync_copy(ones_hbm, ones_b)
        pltpu.make_async_copy(dst_hbm.at[pl.ds(wid * nch, nch)], dsts, s0).wait()
        plsc.subcore_barrier()

        pltpu.async_copy(ones_b, acc.at[dsts.at[0]], s1, add=True)

        def body(j, cr):
            pltpu.async_copy(ones_b, acc.at[dsts.at[j]], s1, add=True)
            pltpu.make_async_copy(ones_b, acc.at[dsts.at[0]], s1).wait()
            return cr

        lax.fori_loop(1, nch, body, 0, unroll=False)
        pltpu.make_async_copy(ones_b, acc.at[dsts.at[0]], s1).wait()

        plsc.subcore_barrier()
        pltpu.sync_copy(acc.at[pl.ds(r0, ROWS_PER_TILE)],
                        out_hbm.at[c, pl.ds(r0, ROWS_PER_TILE)])

    return deg


BR = 2000
_GRID = N // BR


def _l2n(h):
    return h / (jnp.sqrt(jnp.sum(h * h, axis=1, keepdims=True)) + 1e-8)


def _enc_body(x_ref, w1_ref, b1_ref, degp_ref, h_ref, hp_ref, a_ref):
    h = jnp.maximum(
        jnp.dot(x_ref[...], w1_ref[...], preferred_element_type=jnp.float32)
        + b1_ref[...], 0.0)
    h = _l2n(h)
    a = lax.rsqrt(degp_ref[0, :, 0:1] + degp_ref[1, :, 0:1] + 1.0)
    h_ref[...] = h
    a_ref[...] = jnp.broadcast_to(a, h.shape)
    hp_ref[...] = h * a


def _gru_body(p_ref, h_ref, a_ref, wih_ref, whh_ref, bih_ref, bhh_ref, z_ref,
              hn_ref, hpn_ref, zn_ref):
    a = a_ref[...]
    h = h_ref[...]
    agg = (p_ref[0] + p_ref[1]) * a
    gi = jnp.dot(agg, wih_ref[...], preferred_element_type=jnp.float32) + bih_ref[...]
    gh = jnp.dot(h, whh_ref[...], preferred_element_type=jnp.float32) + bhh_ref[...]
    r = jax.nn.sigmoid(gi[:, :D] + gh[:, :D])
    zg = jax.nn.sigmoid(gi[:, D:2 * D] + gh[:, D:2 * D])
    n = jnp.tanh(gi[:, 2 * D:] + r * gh[:, 2 * D:])
    hn = _l2n((1.0 - zg) * n + zg * h)
    hn_ref[...] = hn
    hpn_ref[...] = hn * a
    zn_ref[...] = jnp.maximum(z_ref[...], hn)


def _head_body(z_ref, w2_ref, b2_ref, out_ref):
    out_ref[...] = (
        jnp.dot(z_ref[...], w2_ref[...], preferred_element_type=jnp.float32)
        + b2_ref[...])


def _row_spec(width):
    return pl.BlockSpec((BR, width), lambda i: (i, 0))


def _full_spec(shape):
    return pl.BlockSpec(shape, lambda i: tuple(0 for _ in shape))


_encoder = pl.pallas_call(
    _enc_body,
    grid=(_GRID,),
    in_specs=[
        _row_spec(D),
        _full_spec((D, D)),
        _full_spec((1, D)),
        pl.BlockSpec((NC, BR, DW), lambda i: (0, i, 0)),
    ],
    out_specs=[_row_spec(D), _row_spec(D), _row_spec(D)],
    out_shape=[jax.ShapeDtypeStruct((N, D), jnp.float32)] * 3,
)

_gru = pl.pallas_call(
    _gru_body,
    grid=(_GRID,),
    in_specs=[
        pl.BlockSpec((NC, BR, D), lambda i: (0, i, 0)),
        _row_spec(D),
        _row_spec(D),
        _full_spec((D, 3 * D)),
        _full_spec((D, 3 * D)),
        _full_spec((1, 3 * D)),
        _full_spec((1, 3 * D)),
        _row_spec(D),
    ],
    out_specs=[_row_spec(D), _row_spec(D), _row_spec(D)],
    out_shape=[jax.ShapeDtypeStruct((N, D), jnp.float32)] * 3,
)

_head = pl.pallas_call(
    _head_body,
    grid=(_GRID,),
    in_specs=[_row_spec(D), _full_spec((D, D)), _full_spec((1, D))],
    out_specs=_row_spec(D),
    out_shape=jax.ShapeDtypeStruct((N, D), jnp.float32),
)


def kernel(x, edge_index, W1, b1, Wih, Whh, bih, bhh, W2, b2):
    E = edge_index.shape[1]
    nch = -(-E // (C * NW))
    nch += -nch % 16
    e_pad = C * NW * nch
    spmm = _make_spmm(nch)
    degk = _make_deg(nch)

    src = edge_index[0].astype(jnp.int32)
    dst = edge_index[1].astype(jnp.int32)
    pad = e_pad - E
    src_p = jnp.concatenate([src, jnp.zeros((pad,), jnp.int32)]).reshape(-1, C)
    dst_p = jnp.concatenate([dst, jnp.full((pad,), N, jnp.int32)]).reshape(-1, C)

    zeros_nd = jnp.zeros((NP, D), jnp.float32)
    ones_cw = jnp.ones((C, DW), jnp.float32)

    degp = degk(dst_p, ones_cw, zeros_nd)
    h, hp, a = _encoder(x, W1, b1.reshape(1, D), degp)
    z = h
    bih2 = bih.reshape(1, 3 * D)
    bhh2 = bhh.reshape(1, 3 * D)
    for _ in range(4):
        p = spmm(hp, src_p, dst_p, zeros_nd)
        h, hp, z = _gru(p, h, a, Wih, Whh, bih2, bhh2, z)
    return _head(z, W2, b2.reshape(1, D))
